# baseline (speedup 1.0000x reference)
import functools

import jax
import jax.numpy as jnp
from jax import lax
from jax.experimental import pallas as pl
from jax.experimental.pallas import tpu as pltpu
from jax.experimental.pallas import tpu_sc as plsc

DIMK = 1024
HIDK = 1536
EK = 8
EPAD = 128
TK = 2048
BMK = 256
NTK = 24
NRK = NTK * BMK
HBK = 512
NHK = HIDK // HBK
NWK = 32
TPWK = TK // NWK
CHK = 32
NEG = -1e30

@functools.lru_cache(maxsize=None)
def _sc_mesh():
    return plsc.VectorSubcoreMesh(core_axis_name="c", subcore_axis_name="s")


def _router_body(x_ref, wr_ref, meta_ref, cnt_ref, aux_ref):
    xb = x_ref[...]
    wr = wr_ref[...]
    logits = lax.dot_general(
        xb, wr, (((1,), (1,)), ((), ())), preferred_element_type=jnp.float32)
    lane = lax.broadcasted_iota(jnp.int32, (TK, EPAD), 1)
    valid = lane < EK
    lm = jnp.where(valid, logits, NEG)
    m1 = jnp.max(lm, axis=1, keepdims=True)
    idx1 = jnp.min(jnp.where(lm == m1, lane, EPAD), axis=1, keepdims=True)
    sel1 = lane == idx1
    lm2 = jnp.where(sel1, NEG, lm)
    m2 = jnp.max(lm2, axis=1, keepdims=True)
    idx2 = jnp.min(jnp.where(lm2 == m2, lane, EPAD), axis=1, keepdims=True)
    sel2 = lane == idx2
    e2 = jnp.exp(m2 - m1)
    p1 = 1.0 / (1.0 + e2)
    p2 = 1.0 - p1

    ex = jnp.where(valid, jnp.exp(lm - m1), 0.0)
    den = jnp.sum(ex, axis=1, keepdims=True)
    p_mean = jnp.sum(ex / den, axis=0) / float(TK)
    fcnt = (jnp.sum(sel1.astype(jnp.float32), axis=0)
            + jnp.sum(sel2.astype(jnp.float32), axis=0))
    lb = 0.01 * EK * jnp.sum(fcnt / float(2 * TK) * p_mean)
    lse = m1[:, 0] + jnp.log(den[:, 0])
    z = 0.001 * jnp.mean(lse * lse)
    aux_ref[0, 0] = lb + z

    oh = sel1.astype(jnp.float32) + sel2.astype(jnp.float32)
    ri = lax.broadcasted_iota(jnp.int32, (BMK, BMK), 0)
    ci = lax.broadcasted_iota(jnp.int32, (BMK, BMK), 1)
    ltri = (ci < ri).astype(jnp.float32)
    run = jnp.zeros((1, EPAD), jnp.float32)
    crows = []
    for b in range(TK // BMK):
        ohb = oh[b * BMK:(b + 1) * BMK]
        crows.append(lax.dot_general(
            ltri, ohb, (((1,), (0,)), ((), ())),
            preferred_element_type=jnp.float32) + run)
        run = run + jnp.sum(ohb, axis=0, keepdims=True)
    cum = jnp.concatenate(crows, axis=0)
    counts = run
    pc = jnp.ceil(counts / BMK) * BMK
    ui = lax.broadcasted_iota(jnp.int32, (EPAD, EPAD), 0)
    uj = lax.broadcasted_iota(jnp.int32, (EPAD, EPAD), 1)
    ut = (ui < uj).astype(jnp.float32)
    po = lax.dot_general(pc, ut, (((1,), (0,)), ((), ())),
                         preferred_element_type=jnp.float32)
    posc = cum + po
    pos1 = jnp.sum(jnp.where(sel1, posc, 0.0), axis=1, keepdims=True)
    pos2 = jnp.sum(jnp.where(sel2, posc, 0.0), axis=1, keepdims=True)

    meta = (jnp.where(lane == 0, p1, 0.0) + jnp.where(lane == 1, p2, 0.0)
            + jnp.where(lane == 2, pos1, 0.0)
            + jnp.where(lane == 3, pos2, 0.0))
    meta_ref[...] = meta
    cnt_ref[...] = jnp.broadcast_to(counts, (8, EPAD))


def _gemm_body(te_ref, ti_ref, nt_ref, xs_ref, w1_ref, w3_ref, w2_ref, o_ref):
    i = pl.program_id(0)
    hb = pl.program_id(1)

    @pl.when(i < nt_ref[0])
    def _():
        @pl.when(hb == 0)
        def _z():
            o_ref[...] = jnp.zeros_like(o_ref)

        xb = xs_ref[...].astype(jnp.bfloat16)
        a = lax.dot_general(xb, w1_ref[0], (((1,), (1,)), ((), ())),
                            preferred_element_type=jnp.float32)
        b = lax.dot_general(xb, w3_ref[0], (((1,), (1,)), ((), ())),
                            preferred_element_type=jnp.float32)
        h = (a * jax.nn.sigmoid(a) * b).astype(jnp.bfloat16)
        o_ref[...] += lax.dot_general(h, w2_ref[0], (((1,), (1,)), ((), ())),
                                      preferred_element_type=jnp.float32)


def _scatter_body(x_hbm, p0_hbm, p1_hbm, xs_hbm, i0_v, i1_v, rows_v, sem):
    wid = lax.axis_index("s") * 2 + lax.axis_index("c")
    base = wid * TPWK
    pltpu.sync_copy(x_hbm.at[pl.ds(base, TPWK)], rows_v)
    pltpu.sync_copy(p0_hbm.at[wid], i0_v)
    pltpu.sync_copy(p1_hbm.at[wid], i1_v)
    pltpu.async_copy(rows_v, xs_hbm.at[i0_v], sem).wait()
    pltpu.async_copy(rows_v, xs_hbm.at[i1_v], sem).wait()


def _combine_body(o_hbm, p0_hbm, p1_hbm, g0_hbm, g1_hbm, y_hbm,
                  i0_v, i1_v, g0_v, g1_v, r0_v, r1_v, y_v, s0, s1):
    wid = lax.axis_index("s") * 2 + lax.axis_index("c")
    base = wid * TPWK
    pltpu.sync_copy(p0_hbm.at[wid], i0_v)
    pltpu.sync_copy(p1_hbm.at[wid], i1_v)
    pltpu.sync_copy(g0_hbm.at[wid], g0_v)
    pltpu.sync_copy(g1_hbm.at[wid], g1_v)
    for c in range(TPWK // CHK):
        c0 = c * CHK
        cp0 = pltpu.async_copy(o_hbm.at[i0_v.at[pl.ds(c0, CHK)]], r0_v, s0)
        cp1 = pltpu.async_copy(o_hbm.at[i1_v.at[pl.ds(c0, CHK)]], r1_v, s1)
        cp0.wait()
        cp1.wait()

        def tbody(t, carry):
            g0s = g0_v[c0 + t]
            g1s = g1_v[c0 + t]

            def fbody(f, carry2):
                off = f * 16
                r0 = r0_v[t, pl.ds(off, 16)]
                r1 = r1_v[t, pl.ds(off, 16)]
                y_v[t, pl.ds(off, 16)] = g0s * r0 + g1s * r1
                return carry2

            return lax.fori_loop(0, DIMK // 16, fbody, carry)

        lax.fori_loop(0, CHK, tbody, 0)
        pltpu.sync_copy(y_v, y_hbm.at[pl.ds(base + c0, CHK)])


@functools.lru_cache(maxsize=None)
def _scatter_kernel():
    return pl.kernel(
        _scatter_body,
        out_type=jax.ShapeDtypeStruct((NRK, DIMK), jnp.float32),
        mesh=_sc_mesh(),
        scratch_types=[
            pltpu.VMEM((TPWK,), jnp.int32),
            pltpu.VMEM((TPWK,), jnp.int32),
            pltpu.VMEM((TPWK, DIMK), jnp.float32),
            pltpu.SemaphoreType.DMA,
        ],
    )


@functools.lru_cache(maxsize=None)
def _combine_kernel():
    return pl.kernel(
        _combine_body,
        out_type=jax.ShapeDtypeStruct((TK, DIMK), jnp.float32),
        mesh=_sc_mesh(),
        scratch_types=[
            pltpu.VMEM((TPWK,), jnp.int32),
            pltpu.VMEM((TPWK,), jnp.int32),
            pltpu.VMEM((TPWK, 16), jnp.float32),
            pltpu.VMEM((TPWK, 16), jnp.float32),
            pltpu.VMEM((CHK, DIMK), jnp.float32),
            pltpu.VMEM((CHK, DIMK), jnp.float32),
            pltpu.VMEM((CHK, DIMK), jnp.float32),
            pltpu.SemaphoreType.DMA,
            pltpu.SemaphoreType.DMA,
        ],
    )


def _router_call(xf, wrp):
    return pl.pallas_call(
        _router_body,
        out_shape=[
            jax.ShapeDtypeStruct((TK, EPAD), jnp.float32),
            jax.ShapeDtypeStruct((8, EPAD), jnp.float32),
            jax.ShapeDtypeStruct((1, 1), jnp.float32),
        ],
        in_specs=[
            pl.BlockSpec((TK, DIMK), lambda: (0, 0)),
            pl.BlockSpec((EPAD, DIMK), lambda: (0, 0)),
        ],
        out_specs=[
            pl.BlockSpec((TK, EPAD), lambda: (0, 0)),
            pl.BlockSpec((8, EPAD), lambda: (0, 0)),
            pl.BlockSpec(memory_space=pltpu.SMEM),
        ],
    )(xf, wrp)


def _gemm_call(te, ti, ntv, xs, W1, W3, W2):
    grid_spec = pltpu.PrefetchScalarGridSpec(
        num_scalar_prefetch=3,
        grid=(NTK, NHK),
        in_specs=[
            pl.BlockSpec((BMK, DIMK), lambda i, hb, te, ti, nt: (ti[i], 0)),
            pl.BlockSpec((1, HBK, DIMK), lambda i, hb, te, ti, nt: (te[i], hb, 0)),
            pl.BlockSpec((1, HBK, DIMK), lambda i, hb, te, ti, nt: (te[i], hb, 0)),
            pl.BlockSpec((1, DIMK, HBK), lambda i, hb, te, ti, nt: (te[i], 0, hb)),
        ],
        out_specs=pl.BlockSpec((BMK, DIMK), lambda i, hb, te, ti, nt: (ti[i], 0)),
    )
    return pl.pallas_call(
        _gemm_body,
        grid_spec=grid_spec,
        out_shape=jax.ShapeDtypeStruct((NRK, DIMK), jnp.float32),
    )(te, ti, ntv, xs, W1, W3, W2)


@jax.jit
def kernel(x, Wr, W1, W2, W3):
    B, S, D = x.shape
    T = B * S
    xf = x.reshape(T, D)
    wrp = jnp.zeros((EPAD, D), jnp.float32).at[:EK, :].set(Wr)

    meta, cnts, aux = _router_call(xf, wrp)

    counts = cnts[0, :EK]
    pc = jnp.ceil(counts / BMK) * BMK
    po = jnp.cumsum(pc) - pc
    starts = (po / BMK).astype(jnp.int32)
    n_tiles = (jnp.sum(pc) / BMK).astype(jnp.int32)
    ii = jnp.arange(NTK, dtype=jnp.int32)
    te = jnp.sum((ii[:, None] >= starts[None, :]).astype(jnp.int32), axis=1) - 1
    last_e = jnp.take(te, n_tiles - 1)
    te = jnp.where(ii < n_tiles, te, last_e)
    ti = jnp.minimum(ii, n_tiles - 1)
    ntv = n_tiles[None]

    g0 = jnp.broadcast_to(meta[:, 0].reshape(NWK, TPWK, 1), (NWK, TPWK, 16))
    g1 = jnp.broadcast_to(meta[:, 1].reshape(NWK, TPWK, 1), (NWK, TPWK, 16))
    pos0 = meta[:, 2].astype(jnp.int32).reshape(NWK, TPWK)
    pos1 = meta[:, 3].astype(jnp.int32).reshape(NWK, TPWK)

    xs = _scatter_kernel()(xf, pos0, pos1)
    w1b = W1.astype(jnp.bfloat16)
    w3b = W3.astype(jnp.bfloat16)
    w2b = W2.astype(jnp.bfloat16)
    outs = _gemm_call(te, ti, ntv, xs, w1b, w3b, w2b)
    y = _combine_kernel()(outs, pos0, pos1, g0, g1)

    return y.reshape(B, S, D), aux[0, 0]


# speedup vs baseline: 1.1221x; 1.1221x over previous
import functools

import jax
import jax.numpy as jnp
from jax import lax
from jax.experimental import pallas as pl
from jax.experimental.pallas import tpu as pltpu
from jax.experimental.pallas import tpu_sc as plsc

DIMK = 1024
HIDK = 1536
EK = 8
EPAD = 128
TK = 2048
BMK = 256
NTK = 24
NRK = NTK * BMK
HBK = 512
NHK = HIDK // HBK
NWK = 32
TPWK = TK // NWK
CHK = 32
NEG = -1e30

@functools.lru_cache(maxsize=None)
def _sc_mesh():
    return plsc.VectorSubcoreMesh(core_axis_name="c", subcore_axis_name="s")


def _router_body(x_ref, wr_ref, meta_ref, cnt_ref, aux_ref):
    xb = x_ref[...]
    wr = wr_ref[...]
    logits = lax.dot_general(
        xb, wr, (((1,), (1,)), ((), ())), preferred_element_type=jnp.float32)
    lane = lax.broadcasted_iota(jnp.int32, (TK, EPAD), 1)
    valid = lane < EK
    lm = jnp.where(valid, logits, NEG)
    m1 = jnp.max(lm, axis=1, keepdims=True)
    idx1 = jnp.min(jnp.where(lm == m1, lane, EPAD), axis=1, keepdims=True)
    sel1 = lane == idx1
    lm2 = jnp.where(sel1, NEG, lm)
    m2 = jnp.max(lm2, axis=1, keepdims=True)
    idx2 = jnp.min(jnp.where(lm2 == m2, lane, EPAD), axis=1, keepdims=True)
    sel2 = lane == idx2
    e2 = jnp.exp(m2 - m1)
    p1 = 1.0 / (1.0 + e2)
    p2 = 1.0 - p1

    ex = jnp.where(valid, jnp.exp(lm - m1), 0.0)
    den = jnp.sum(ex, axis=1, keepdims=True)
    p_mean = jnp.sum(ex / den, axis=0) / float(TK)
    fcnt = (jnp.sum(sel1.astype(jnp.float32), axis=0)
            + jnp.sum(sel2.astype(jnp.float32), axis=0))
    lb = 0.01 * EK * jnp.sum(fcnt / float(2 * TK) * p_mean)
    lse = m1[:, 0] + jnp.log(den[:, 0])
    z = 0.001 * jnp.mean(lse * lse)
    aux_ref[0, 0] = lb + z

    oh = sel1.astype(jnp.float32) + sel2.astype(jnp.float32)
    ri = lax.broadcasted_iota(jnp.int32, (BMK, BMK), 0)
    ci = lax.broadcasted_iota(jnp.int32, (BMK, BMK), 1)
    ltri = (ci < ri).astype(jnp.float32)
    run = jnp.zeros((1, EPAD), jnp.float32)
    crows = []
    for b in range(TK // BMK):
        ohb = oh[b * BMK:(b + 1) * BMK]
        crows.append(lax.dot_general(
            ltri, ohb, (((1,), (0,)), ((), ())),
            preferred_element_type=jnp.float32) + run)
        run = run + jnp.sum(ohb, axis=0, keepdims=True)
    cum = jnp.concatenate(crows, axis=0)
    counts = run
    pc = jnp.ceil(counts / BMK) * BMK
    ui = lax.broadcasted_iota(jnp.int32, (EPAD, EPAD), 0)
    uj = lax.broadcasted_iota(jnp.int32, (EPAD, EPAD), 1)
    ut = (ui < uj).astype(jnp.float32)
    po = lax.dot_general(pc, ut, (((1,), (0,)), ((), ())),
                         preferred_element_type=jnp.float32)
    posc = cum + po
    pos1 = jnp.sum(jnp.where(sel1, posc, 0.0), axis=1, keepdims=True)
    pos2 = jnp.sum(jnp.where(sel2, posc, 0.0), axis=1, keepdims=True)

    meta = (jnp.where(lane == 0, p1, 0.0) + jnp.where(lane == 1, p2, 0.0)
            + jnp.where(lane == 2, pos1, 0.0)
            + jnp.where(lane == 3, pos2, 0.0))
    meta_ref[...] = meta
    cnt_ref[...] = jnp.broadcast_to(counts, (8, EPAD))


def _gemm_body(te_ref, ti_ref, nt_ref, xs_ref, w1_ref, w3_ref, w2_ref, o_ref):
    i = pl.program_id(0)
    hb = pl.program_id(1)

    @pl.when(i < nt_ref[0])
    def _():
        @pl.when(hb == 0)
        def _z():
            o_ref[...] = jnp.zeros_like(o_ref)

        xb = xs_ref[...].astype(jnp.bfloat16)
        w1 = w1_ref[0].astype(jnp.bfloat16)
        w3 = w3_ref[0].astype(jnp.bfloat16)
        w2 = w2_ref[0].astype(jnp.bfloat16)
        a = lax.dot_general(xb, w1, (((1,), (1,)), ((), ())),
                            preferred_element_type=jnp.float32)
        b = lax.dot_general(xb, w3, (((1,), (1,)), ((), ())),
                            preferred_element_type=jnp.float32)
        h = (a * jax.nn.sigmoid(a) * b).astype(jnp.bfloat16)
        o_ref[...] += lax.dot_general(h, w2, (((1,), (1,)), ((), ())),
                                      preferred_element_type=jnp.float32)


def _scatter_body(x_hbm, p0_hbm, p1_hbm, xs_hbm, i0_v, i1_v, rows_v, sem):
    wid = lax.axis_index("s") * 2 + lax.axis_index("c")
    base = wid * TPWK
    pltpu.sync_copy(x_hbm.at[pl.ds(base, TPWK)], rows_v)
    pltpu.sync_copy(p0_hbm.at[wid], i0_v)
    pltpu.sync_copy(p1_hbm.at[wid], i1_v)
    pltpu.async_copy(rows_v, xs_hbm.at[i0_v], sem).wait()
    pltpu.async_copy(rows_v, xs_hbm.at[i1_v], sem).wait()


def _combine_body(o_hbm, p0_hbm, p1_hbm, g0_hbm, g1_hbm, y_hbm,
                  i0_v, i1_v, g0_v, g1_v, r0_v, r1_v, y_v, s0, s1):
    wid = lax.axis_index("s") * 2 + lax.axis_index("c")
    base = wid * TPWK
    pltpu.sync_copy(p0_hbm.at[wid], i0_v)
    pltpu.sync_copy(p1_hbm.at[wid], i1_v)
    pltpu.sync_copy(g0_hbm.at[wid], g0_v)
    pltpu.sync_copy(g1_hbm.at[wid], g1_v)
    for c in range(TPWK // CHK):
        c0 = c * CHK
        cp0 = pltpu.async_copy(o_hbm.at[i0_v.at[pl.ds(c0, CHK)]], r0_v, s0)
        cp1 = pltpu.async_copy(o_hbm.at[i1_v.at[pl.ds(c0, CHK)]], r1_v, s1)
        cp0.wait()
        cp1.wait()

        def tbody(t, carry):
            g0s = g0_v[c0 + t]
            g1s = g1_v[c0 + t]

            def fbody(f, carry2):
                off = f * 16
                r0 = r0_v[t, pl.ds(off, 16)]
                r1 = r1_v[t, pl.ds(off, 16)]
                y_v[t, pl.ds(off, 16)] = g0s * r0 + g1s * r1
                return carry2

            return lax.fori_loop(0, DIMK // 16, fbody, carry)

        lax.fori_loop(0, CHK, tbody, 0)
        pltpu.sync_copy(y_v, y_hbm.at[pl.ds(base + c0, CHK)])


@functools.lru_cache(maxsize=None)
def _scatter_kernel():
    return pl.kernel(
        _scatter_body,
        out_type=jax.ShapeDtypeStruct((NRK, DIMK), jnp.float32),
        mesh=_sc_mesh(),
        scratch_types=[
            pltpu.VMEM((TPWK,), jnp.int32),
            pltpu.VMEM((TPWK,), jnp.int32),
            pltpu.VMEM((TPWK, DIMK), jnp.float32),
            pltpu.SemaphoreType.DMA,
        ],
    )


@functools.lru_cache(maxsize=None)
def _combine_kernel():
    return pl.kernel(
        _combine_body,
        out_type=jax.ShapeDtypeStruct((TK, DIMK), jnp.float32),
        mesh=_sc_mesh(),
        scratch_types=[
            pltpu.VMEM((TPWK,), jnp.int32),
            pltpu.VMEM((TPWK,), jnp.int32),
            pltpu.VMEM((TPWK, 16), jnp.float32),
            pltpu.VMEM((TPWK, 16), jnp.float32),
            pltpu.VMEM((CHK, DIMK), jnp.float32),
            pltpu.VMEM((CHK, DIMK), jnp.float32),
            pltpu.VMEM((CHK, DIMK), jnp.float32),
            pltpu.SemaphoreType.DMA,
            pltpu.SemaphoreType.DMA,
        ],
    )


def _router_call(xf, wrp):
    return pl.pallas_call(
        _router_body,
        out_shape=[
            jax.ShapeDtypeStruct((TK, EPAD), jnp.float32),
            jax.ShapeDtypeStruct((8, EPAD), jnp.float32),
            jax.ShapeDtypeStruct((1, 1), jnp.float32),
        ],
        in_specs=[
            pl.BlockSpec((TK, DIMK), lambda: (0, 0)),
            pl.BlockSpec((EPAD, DIMK), lambda: (0, 0)),
        ],
        out_specs=[
            pl.BlockSpec((TK, EPAD), lambda: (0, 0)),
            pl.BlockSpec((8, EPAD), lambda: (0, 0)),
            pl.BlockSpec(memory_space=pltpu.SMEM),
        ],
    )(xf, wrp)


def _gemm_call(te, ti, ntv, xs, W1, W3, W2):
    grid_spec = pltpu.PrefetchScalarGridSpec(
        num_scalar_prefetch=3,
        grid=(NTK, NHK),
        in_specs=[
            pl.BlockSpec((BMK, DIMK), lambda i, hb, te, ti, nt: (ti[i], 0)),
            pl.BlockSpec((1, HBK, DIMK), lambda i, hb, te, ti, nt: (te[i], hb, 0)),
            pl.BlockSpec((1, HBK, DIMK), lambda i, hb, te, ti, nt: (te[i], hb, 0)),
            pl.BlockSpec((1, DIMK, HBK), lambda i, hb, te, ti, nt: (te[i], 0, hb)),
        ],
        out_specs=pl.BlockSpec((BMK, DIMK), lambda i, hb, te, ti, nt: (ti[i], 0)),
    )
    return pl.pallas_call(
        _gemm_body,
        grid_spec=grid_spec,
        out_shape=jax.ShapeDtypeStruct((NRK, DIMK), jnp.float32),
    )(te, ti, ntv, xs, W1, W3, W2)


@jax.jit
def kernel(x, Wr, W1, W2, W3):
    B, S, D = x.shape
    T = B * S
    xf = x.reshape(T, D)
    wrp = jnp.zeros((EPAD, D), jnp.float32).at[:EK, :].set(Wr)

    meta, cnts, aux = _router_call(xf, wrp)

    counts = cnts[0, :EK]
    pc = jnp.ceil(counts / BMK) * BMK
    po = jnp.cumsum(pc) - pc
    starts = (po / BMK).astype(jnp.int32)
    n_tiles = (jnp.sum(pc) / BMK).astype(jnp.int32)
    ii = jnp.arange(NTK, dtype=jnp.int32)
    te = jnp.sum((ii[:, None] >= starts[None, :]).astype(jnp.int32), axis=1) - 1
    last_e = jnp.take(te, n_tiles - 1)
    te = jnp.where(ii < n_tiles, te, last_e)
    ti = jnp.minimum(ii, n_tiles - 1)
    ntv = n_tiles[None]

    g0 = jnp.broadcast_to(meta[:, 0].reshape(NWK, TPWK, 1), (NWK, TPWK, 16))
    g1 = jnp.broadcast_to(meta[:, 1].reshape(NWK, TPWK, 1), (NWK, TPWK, 16))
    pos0 = meta[:, 2].astype(jnp.int32).reshape(NWK, TPWK)
    pos1 = meta[:, 3].astype(jnp.int32).reshape(NWK, TPWK)

    xs = _scatter_kernel()(xf, pos0, pos1)
    outs = _gemm_call(te, ti, ntv, xs, W1, W3, W2)
    y = _combine_kernel()(outs, pos0, pos1, g0, g1)

    return y.reshape(B, S, D), aux[0, 0]


# speedup vs baseline: 1.6732x; 1.4911x over previous
import functools

import jax
import jax.numpy as jnp
from jax import lax
from jax.experimental import pallas as pl
from jax.experimental.pallas import tpu as pltpu
from jax.experimental.pallas import tpu_sc as plsc

DIMK = 1024
HIDK = 1536
EK = 8
TK = 2048
BMK = 256
NTK = 24
NRK = NTK * BMK
NWK = 32
TPWK = TK // NWK
CHK = 32
NEG = -1e30

@functools.lru_cache(maxsize=None)
def _sc_mesh():
    return plsc.VectorSubcoreMesh(core_axis_name="c", subcore_axis_name="s")


def _router_body(x_ref, wr_ref, g0_ref, g1_ref, p0_ref, p1_ref, cnt_ref,
                 aux_ref):
    xb = x_ref[...]
    wr = wr_ref[...]
    lm = lax.dot_general(
        xb, wr, (((1,), (1,)), ((), ())), preferred_element_type=jnp.float32)
    lane = lax.broadcasted_iota(jnp.int32, (TK, EK), 1)
    m1 = jnp.max(lm, axis=1, keepdims=True)
    idx1 = jnp.min(jnp.where(lm == m1, lane, EK), axis=1, keepdims=True)
    sel1 = lane == idx1
    lm2 = jnp.where(sel1, NEG, lm)
    m2 = jnp.max(lm2, axis=1, keepdims=True)
    idx2 = jnp.min(jnp.where(lm2 == m2, lane, EK), axis=1, keepdims=True)
    sel2 = lane == idx2
    e2 = jnp.exp(m2 - m1)
    p1 = 1.0 / (1.0 + e2)
    p2 = 1.0 - p1

    ex = jnp.exp(lm - m1)
    den = jnp.sum(ex, axis=1, keepdims=True)
    p_mean = jnp.sum(ex / den, axis=0) / float(TK)
    fcnt = (jnp.sum(sel1.astype(jnp.float32), axis=0)
            + jnp.sum(sel2.astype(jnp.float32), axis=0))
    lb = 0.01 * EK * jnp.sum(fcnt / float(2 * TK) * p_mean)
    lse = m1[:, 0] + jnp.log(den[:, 0])
    z = 0.001 * jnp.mean(lse * lse)
    aux_ref[0, 0] = lb + z

    oh = sel1.astype(jnp.float32) + sel2.astype(jnp.float32)
    ri = lax.broadcasted_iota(jnp.int32, (BMK, BMK), 0)
    ci = lax.broadcasted_iota(jnp.int32, (BMK, BMK), 1)
    ltri = (ci < ri).astype(jnp.float32)
    run = jnp.zeros((1, EK), jnp.float32)
    crows = []
    for b in range(TK // BMK):
        ohb = oh[b * BMK:(b + 1) * BMK]
        crows.append(lax.dot_general(
            ltri, ohb, (((1,), (0,)), ((), ())),
            preferred_element_type=jnp.float32) + run)
        run = run + jnp.sum(ohb, axis=0, keepdims=True)
    cum = jnp.concatenate(crows, axis=0)
    counts = run
    pc = jnp.ceil(counts / BMK) * BMK
    ui = lax.broadcasted_iota(jnp.int32, (EK, EK), 0)
    uj = lax.broadcasted_iota(jnp.int32, (EK, EK), 1)
    ut = (ui < uj).astype(jnp.float32)
    po = lax.dot_general(pc, ut, (((1,), (0,)), ((), ())),
                         preferred_element_type=jnp.float32)
    posc = cum + po
    pos1 = jnp.sum(jnp.where(sel1, posc, 0.0), axis=1, keepdims=True)
    pos2 = jnp.sum(jnp.where(sel2, posc, 0.0), axis=1, keepdims=True)

    cnt_ref[...] = jnp.broadcast_to(counts, (8, EK))

    g0_ref[...] = jnp.broadcast_to(p1, (TK, 16))
    g1_ref[...] = jnp.broadcast_to(p2, (TK, 16))
    p0_ref[...] = pos1.astype(jnp.int32)
    p1_ref[...] = pos2.astype(jnp.int32)


def _ffn_tile(xs_ref, w1_ref, w3_ref, w2_ref, o_ref):
    xb = xs_ref[...].astype(jnp.bfloat16)
    w1 = w1_ref[0].astype(jnp.bfloat16)
    w3 = w3_ref[0].astype(jnp.bfloat16)
    w2 = w2_ref[0].astype(jnp.bfloat16)
    a = lax.dot_general(xb, w1, (((1,), (1,)), ((), ())),
                        preferred_element_type=jnp.float32)
    b = lax.dot_general(xb, w3, (((1,), (1,)), ((), ())),
                        preferred_element_type=jnp.float32)
    h = (a * jax.nn.sigmoid(a) * b).astype(jnp.bfloat16)
    o_ref[...] = lax.dot_general(h, w2, (((1,), (1,)), ((), ())),
                                 preferred_element_type=jnp.float32)


def _gemm_body(te_ref, ti_ref, nt_ref, xs_ref, w1_ref, w3_ref, w2_ref,
               o_ref):
    i = pl.program_id(0)

    @pl.when(i < nt_ref[0])
    def _():
        _ffn_tile(xs_ref, w1_ref, w3_ref, w2_ref, o_ref)


def _scatter_body(x_hbm, p0_hbm, p1_hbm, xs_hbm, i0_v, i1_v, rows_v, s0, s1):
    wid = lax.axis_index("s") * 2 + lax.axis_index("c")
    base = wid * TPWK
    pltpu.sync_copy(x_hbm.at[pl.ds(base, TPWK)], rows_v)
    pltpu.sync_copy(p0_hbm.at[wid], i0_v)
    pltpu.sync_copy(p1_hbm.at[wid], i1_v)
    pltpu.async_copy(rows_v, xs_hbm.at[i0_v], s0).wait()
    pltpu.async_copy(rows_v, xs_hbm.at[i1_v], s1).wait()


def _combine_body(o_hbm, p0_hbm, p1_hbm, g0_hbm, g1_hbm, y_hbm,
                  i0_v, i1_v, g0_v, g1_v, r0_v, r1_v, y_v, s0, s1):
    wid = lax.axis_index("s") * 2 + lax.axis_index("c")
    base = wid * TPWK
    pltpu.sync_copy(p0_hbm.at[wid], i0_v)
    pltpu.sync_copy(p1_hbm.at[wid], i1_v)
    pltpu.sync_copy(g0_hbm.at[wid], g0_v)
    pltpu.sync_copy(g1_hbm.at[wid], g1_v)
    for c in range(TPWK // CHK):
        c0 = c * CHK
        cp0 = pltpu.async_copy(o_hbm.at[i0_v.at[pl.ds(c0, CHK)]], r0_v, s0)
        cp1 = pltpu.async_copy(o_hbm.at[i1_v.at[pl.ds(c0, CHK)]], r1_v, s1)
        cp0.wait()
        cp1.wait()

        def tbody(t, carry):
            g0s = g0_v[c0 + t]
            g1s = g1_v[c0 + t]

            def fbody(f, carry2):
                for u in range(4):
                    off = f * 64 + u * 16
                    r0 = r0_v[t, pl.ds(off, 16)]
                    r1 = r1_v[t, pl.ds(off, 16)]
                    y_v[t, pl.ds(off, 16)] = g0s * r0 + g1s * r1
                return carry2

            return lax.fori_loop(0, DIMK // 64, fbody, carry)

        lax.fori_loop(0, CHK, tbody, 0)
        pltpu.sync_copy(y_v, y_hbm.at[pl.ds(base + c0, CHK)])


@functools.lru_cache(maxsize=None)
def _scatter_kernel():
    return pl.kernel(
        _scatter_body,
        out_type=jax.ShapeDtypeStruct((NRK, DIMK), jnp.float32),
        mesh=_sc_mesh(),
        scratch_types=[
            pltpu.VMEM((TPWK,), jnp.int32),
            pltpu.VMEM((TPWK,), jnp.int32),
            pltpu.VMEM((TPWK, DIMK), jnp.float32),
            pltpu.SemaphoreType.DMA,
            pltpu.SemaphoreType.DMA,
        ],
    )


@functools.lru_cache(maxsize=None)
def _combine_kernel():
    return pl.kernel(
        _combine_body,
        out_type=jax.ShapeDtypeStruct((TK, DIMK), jnp.float32),
        mesh=_sc_mesh(),
        scratch_types=[
            pltpu.VMEM((TPWK,), jnp.int32),
            pltpu.VMEM((TPWK,), jnp.int32),
            pltpu.VMEM((TPWK, 16), jnp.float32),
            pltpu.VMEM((TPWK, 16), jnp.float32),
            pltpu.VMEM((CHK, DIMK), jnp.float32),
            pltpu.VMEM((CHK, DIMK), jnp.float32),
            pltpu.VMEM((CHK, DIMK), jnp.float32),
            pltpu.SemaphoreType.DMA,
            pltpu.SemaphoreType.DMA,
        ],
    )


def _router_call(xf, wrp):
    return pl.pallas_call(
        _router_body,
        out_shape=[
            jax.ShapeDtypeStruct((TK, 16), jnp.float32),
            jax.ShapeDtypeStruct((TK, 16), jnp.float32),
            jax.ShapeDtypeStruct((TK, 1), jnp.int32),
            jax.ShapeDtypeStruct((TK, 1), jnp.int32),
            jax.ShapeDtypeStruct((8, EK), jnp.float32),
            jax.ShapeDtypeStruct((1, 1), jnp.float32),
        ],
        in_specs=[
            pl.BlockSpec((TK, DIMK), lambda: (0, 0)),
            pl.BlockSpec((EK, DIMK), lambda: (0, 0)),
        ],
        out_specs=[
            pl.BlockSpec((TK, 16), lambda: (0, 0)),
            pl.BlockSpec((TK, 16), lambda: (0, 0)),
            pl.BlockSpec((TK, 1), lambda: (0, 0)),
            pl.BlockSpec((TK, 1), lambda: (0, 0)),
            pl.BlockSpec((8, EK), lambda: (0, 0)),
            pl.BlockSpec(memory_space=pltpu.SMEM),
        ],
    )(xf, wrp)


def _gemm_call(te, ti, ntv, xs, W1, W3, W2):
    grid_spec = pltpu.PrefetchScalarGridSpec(
        num_scalar_prefetch=3,
        grid=(NTK,),
        in_specs=[
            pl.BlockSpec((BMK, DIMK), lambda i, te, ti, nt: (ti[i], 0)),
            pl.BlockSpec((1, HIDK, DIMK), lambda i, te, ti, nt: (te[i], 0, 0)),
            pl.BlockSpec((1, HIDK, DIMK), lambda i, te, ti, nt: (te[i], 0, 0)),
            pl.BlockSpec((1, DIMK, HIDK), lambda i, te, ti, nt: (te[i], 0, 0)),
        ],
        out_specs=pl.BlockSpec((BMK, DIMK), lambda i, te, ti, nt: (ti[i], 0)),
    )
    return pl.pallas_call(
        _gemm_body,
        grid_spec=grid_spec,
        out_shape=jax.ShapeDtypeStruct((NRK, DIMK), jnp.float32),
    )(te, ti, ntv, xs, W1, W3, W2)


@jax.jit
def kernel(x, Wr, W1, W2, W3):
    B, S, D = x.shape
    T = B * S
    xf = x.reshape(T, D)

    g0m, g1m, p0m, p1m, cnts, aux = _router_call(xf, Wr)

    counts = cnts[0]
    pc = jnp.ceil(counts / BMK) * BMK
    po = jnp.cumsum(pc) - pc
    starts = (po / BMK).astype(jnp.int32)
    n_tiles = (jnp.sum(pc) / BMK).astype(jnp.int32)
    ii = jnp.arange(NTK, dtype=jnp.int32)
    te = jnp.sum((ii[:, None] >= starts[None, :]).astype(jnp.int32), axis=1) - 1
    last_e = jnp.take(te, n_tiles - 1)
    te = jnp.where(ii < n_tiles, te, last_e)
    ti = jnp.minimum(ii, n_tiles - 1)
    ntv = n_tiles[None]

    g0 = g0m.reshape(NWK, TPWK, 16)
    g1 = g1m.reshape(NWK, TPWK, 16)
    pos0 = p0m.reshape(NWK, TPWK)
    pos1 = p1m.reshape(NWK, TPWK)

    xs = _scatter_kernel()(xf, pos0, pos1)
    outs = _gemm_call(te, ti, ntv, xs, W1, W3, W2)
    y = _combine_kernel()(outs, pos0, pos1, g0, g1)

    return y.reshape(B, S, D), aux[0, 0]
